# Initial kernel scaffold; baseline (speedup 1.0000x reference)
import jax
import jax.numpy as jnp
from jax.experimental import pallas as pl


def kernel(x_domain0, x_domain1, x_domain2, x_domain3, x_domain4, ei_d0d0, ei_d1d1, ei_d2d2, ei_d3d3, ei_d1d0, ei_d0d1, ei_d1d2, ei_d2d1, ei_d1d3, ei_d3d1, batch_domain0, batch_domain1, batch_domain2, batch_domain3, batch_domain4, index, Wself0, Wself1, Wm0_0, Wm0_1, Wm0_2, Wm0_3, Wm0_4, Wm0_5, Wm0_6, Wm0_7, Wm0_8, Wm0_9, Wm1_0, Wm1_1, Wm1_2, Wm1_3, Wm1_4, Wm1_5, Wm1_6, Wm1_7, Wm1_8, Wm1_9, Wp_0, Wp_1, Wp_2, Wp_3, Wc, bc):
    raise NotImplementedError("write your pallas kernel here")


# speedup vs baseline: 2.4726x; 2.4726x over previous
import functools

import jax
import jax.numpy as jnp
from jax.experimental import pallas as pl

_N = 4096
_H = 128
_K_TOP = 16
_EDGE = [(0, 0), (1, 1), (2, 2), (3, 3), (1, 0), (0, 1), (1, 2), (2, 1),
         (1, 3), (3, 1)]


def _mm_kern(a_ref, b_ref, o_ref):
    o_ref[...] = jnp.dot(a_ref[...], b_ref[...],
                         preferred_element_type=jnp.float32)


def _matmul(a, b, bm=512):
    m, k = a.shape
    n = b.shape[1]
    return pl.pallas_call(
        _mm_kern,
        grid=(m // bm,),
        in_specs=[pl.BlockSpec((bm, k), lambda i: (i, 0)),
                  pl.BlockSpec((k, n), lambda i: (0, 0))],
        out_specs=pl.BlockSpec((bm, n), lambda i: (i, 0)),
        out_shape=jax.ShapeDtypeStruct((m, n), jnp.float32),
    )(a, b)


def _pmm(a, b, bm=512):
    k = a.shape[1]
    n = b.shape[1]
    kp = ((k + 127) // 128) * 128
    np_ = ((n + 127) // 128) * 128
    if kp != k:
        a = jnp.pad(a, ((0, 0), (0, kp - k)))
        b = jnp.pad(b, ((0, kp - k), (0, 0)))
    if np_ != n:
        b = jnp.pad(b, ((0, 0), (0, np_ - n)))
    r = _matmul(a, b, bm)
    return r[:, :n] if np_ != n else r


def _fused_kern(n_adj, xself_ref, *refs):
    o_ref = refs[-1]
    acc = xself_ref[...]
    for j in range(n_adj):
        acc = acc + jnp.dot(refs[j][...], refs[n_adj + j][...],
                            preferred_element_type=jnp.float32)
    o_ref[...] = jnp.maximum(acc, 0.0)


def _fused(xself, adjs, ps, bm=256):
    n_adj = len(adjs)
    in_specs = [pl.BlockSpec((bm, _H), lambda i: (i, 0))]
    in_specs += [pl.BlockSpec((bm, _N), lambda i: (i, 0))] * n_adj
    in_specs += [pl.BlockSpec((_N, _H), lambda i: (0, 0))] * n_adj
    return pl.pallas_call(
        functools.partial(_fused_kern, n_adj),
        grid=(_N // bm,),
        in_specs=in_specs,
        out_specs=pl.BlockSpec((bm, _H), lambda i: (i, 0)),
        out_shape=jax.ShapeDtypeStruct((_N, _H), jnp.float32),
    )(xself, *adjs, *ps)


def _topk_kern(d_ref, g_ref, o_ref):
    d = d_ref[...]
    edge = d > 0.0
    score = jnp.where(edge, d + g_ref[...], -1e9)
    cur = score
    for _ in range(_K_TOP - 1):
        m = jnp.max(cur, axis=1, keepdims=True)
        cur = jnp.where(cur == m, -3.0e38, cur)
    t = jnp.max(cur, axis=1, keepdims=True)
    o_ref[...] = jnp.where(edge & (score >= t), 1.0, 0.0)


def _topk_select(dense, g, bm=256):
    return pl.pallas_call(
        _topk_kern,
        grid=(_N // bm,),
        in_specs=[pl.BlockSpec((bm, _N), lambda i: (i, 0)),
                  pl.BlockSpec((bm, _N), lambda i: (i, 0))],
        out_specs=pl.BlockSpec((bm, _N), lambda i: (i, 0)),
        out_shape=jax.ShapeDtypeStruct((_N, _N), jnp.float32),
    )(dense, g)


def kernel(x_domain0, x_domain1, x_domain2, x_domain3, x_domain4,
           ei_d0d0, ei_d1d1, ei_d2d2, ei_d3d3, ei_d1d0, ei_d0d1,
           ei_d1d2, ei_d2d1, ei_d1d3, ei_d3d1,
           batch_domain0, batch_domain1, batch_domain2, batch_domain3,
           batch_domain4, index,
           Wself0, Wself1,
           Wm0_0, Wm0_1, Wm0_2, Wm0_3, Wm0_4, Wm0_5, Wm0_6, Wm0_7, Wm0_8,
           Wm0_9,
           Wm1_0, Wm1_1, Wm1_2, Wm1_3, Wm1_4, Wm1_5, Wm1_6, Wm1_7, Wm1_8,
           Wm1_9,
           Wp_0, Wp_1, Wp_2, Wp_3, Wc, bc):
    xs = [x_domain0, x_domain1, x_domain2, x_domain3, x_domain4]
    eis = [ei_d0d0, ei_d1d1, ei_d2d2, ei_d3d3, ei_d1d0, ei_d0d1,
           ei_d1d2, ei_d2d1, ei_d1d3, ei_d3d1]
    wm0 = [Wm0_0, Wm0_1, Wm0_2, Wm0_3, Wm0_4, Wm0_5, Wm0_6, Wm0_7, Wm0_8,
           Wm0_9]
    wm1 = [Wm1_0, Wm1_1, Wm1_2, Wm1_3, Wm1_4, Wm1_5, Wm1_6, Wm1_7, Wm1_8,
           Wm1_9]
    wps = [Wp_0, Wp_1, Wp_2, Wp_3]

    adj = [jnp.zeros((_N, _N), jnp.float32).at[e[1], e[0]].add(1.0)
           for e in eis]

    def embed_layer(h, Wself, Wms, adjs_by_type, types):
        self_p = [_pmm(h[k], Wself) for k in range(5)]
        ps = {i: _pmm(h[_EDGE[i][0]], Wms[i]) for i in types}
        out = []
        for k in range(5):
            inc = [i for i in types if _EDGE[i][1] == k]
            out.append(_fused(self_p[k],
                              [adjs_by_type[i] for i in inc],
                              [ps[i] for i in inc]))
        return out

    def gnn(types, attn_adj=None):
        h = xs
        for Wself, Wms in ((Wself0, wm0), (Wself1, wm1)):
            adjs = {i: (attn_adj[i] if attn_adj is not None and i in attn_adj
                        else adj[i]) for i in types}
            h = embed_layer(h, Wself, Wms, adjs, types)
        return h

    h1 = gnn([0, 1, 2, 3])

    gk = jax.random.key(42)
    attn_adj = {}
    for i in range(4):
        s, d = _EDGE[i]
        ei = eis[i]
        a = _pmm(h1[s], wps[i][:_H])[:, 0]
        b = _pmm(h1[d], wps[i][_H:])[:, 0]
        pred = jax.nn.sigmoid(a[ei[0]] + b[ei[1]])
        dense = jnp.zeros((_N, _N), jnp.float32).at[ei[0], ei[1]].add(pred)
        u = jax.random.uniform(jax.random.fold_in(gk, i), (_N, _N),
                               minval=1e-06, maxval=1.0 - 1e-06)
        g = -jnp.log(-jnp.log(u))
        w_dense = _topk_select(dense, g)
        w_e = w_dense[ei[0], ei[1]]
        attn_adj[i] = jnp.zeros((_N, _N), jnp.float32).at[ei[1], ei[0]].add(
            w_e)

    h2 = gnn(list(range(10)), attn_adj=attn_adj)

    pool = h2[1][index].reshape(1, -1)
    return jax.nn.softmax(pool @ Wc + bc, axis=1)


# speedup vs baseline: 2.7647x; 1.1181x over previous
import functools

import jax
import jax.numpy as jnp
from jax.experimental import pallas as pl

_N = 4096
_H = 128
_K_TOP = 16
_EDGE = [(0, 0), (1, 1), (2, 2), (3, 3), (1, 0), (0, 1), (1, 2), (2, 1),
         (1, 3), (3, 1)]


def _mm_kern(a_ref, b_ref, o_ref):
    o_ref[...] = jnp.dot(a_ref[...], b_ref[...],
                         preferred_element_type=jnp.float32)


def _matmul(a, b, bm=512):
    m, k = a.shape
    n = b.shape[1]
    return pl.pallas_call(
        _mm_kern,
        grid=(m // bm,),
        in_specs=[pl.BlockSpec((bm, k), lambda i: (i, 0)),
                  pl.BlockSpec((k, n), lambda i: (0, 0))],
        out_specs=pl.BlockSpec((bm, n), lambda i: (i, 0)),
        out_shape=jax.ShapeDtypeStruct((m, n), jnp.float32),
    )(a, b)


def _pmm(a, b, bm=512):
    k = a.shape[1]
    n = b.shape[1]
    kp = ((k + 127) // 128) * 128
    np_ = ((n + 127) // 128) * 128
    if kp != k:
        a = jnp.pad(a, ((0, 0), (0, kp - k)))
        b = jnp.pad(b, ((0, kp - k), (0, 0)))
    if np_ != n:
        b = jnp.pad(b, ((0, 0), (0, np_ - n)))
    r = _matmul(a, b, bm)
    return r[:, :n] if np_ != n else r


def _fused_kern(n_adj, xself_ref, *refs):
    o_ref = refs[-1]
    acc = xself_ref[...]
    for j in range(n_adj):
        acc = acc + jnp.dot(refs[j][...], refs[n_adj + j][...],
                            preferred_element_type=jnp.float32)
    o_ref[...] = jnp.maximum(acc, 0.0)


def _fused(xself, adjs, ps, bm=256):
    n_adj = len(adjs)
    in_specs = [pl.BlockSpec((bm, _H), lambda i: (i, 0))]
    in_specs += [pl.BlockSpec((bm, _N), lambda i: (i, 0))] * n_adj
    in_specs += [pl.BlockSpec((_N, _H), lambda i: (0, 0))] * n_adj
    return pl.pallas_call(
        functools.partial(_fused_kern, n_adj),
        grid=(_N // bm,),
        in_specs=in_specs,
        out_specs=pl.BlockSpec((bm, _H), lambda i: (i, 0)),
        out_shape=jax.ShapeDtypeStruct((_N, _H), jnp.float32),
    )(xself, *adjs, *ps)


def _topk_kern(d_ref, g_ref, c_ref, o_ref):
    d = d_ref[...]
    edge = d > 0.0
    score = jnp.where(edge, d + g_ref[...], -1e9)
    cur = score
    for _ in range(_K_TOP - 1):
        m = jnp.max(cur, axis=1, keepdims=True)
        cur = jnp.where(cur == m, -3.0e38, cur)
    t = jnp.max(cur, axis=1, keepdims=True)
    o_ref[...] = jnp.where(edge & (score >= t), c_ref[...], 0.0)


def _topk_select(dense, g, counts, bm=256):
    return pl.pallas_call(
        _topk_kern,
        grid=(_N // bm,),
        in_specs=[pl.BlockSpec((bm, _N), lambda i: (i, 0)),
                  pl.BlockSpec((bm, _N), lambda i: (i, 0)),
                  pl.BlockSpec((bm, _N), lambda i: (i, 0))],
        out_specs=pl.BlockSpec((bm, _N), lambda i: (i, 0)),
        out_shape=jax.ShapeDtypeStruct((_N, _N), jnp.float32),
    )(dense, g, counts)


def kernel(x_domain0, x_domain1, x_domain2, x_domain3, x_domain4,
           ei_d0d0, ei_d1d1, ei_d2d2, ei_d3d3, ei_d1d0, ei_d0d1,
           ei_d1d2, ei_d2d1, ei_d1d3, ei_d3d1,
           batch_domain0, batch_domain1, batch_domain2, batch_domain3,
           batch_domain4, index,
           Wself0, Wself1,
           Wm0_0, Wm0_1, Wm0_2, Wm0_3, Wm0_4, Wm0_5, Wm0_6, Wm0_7, Wm0_8,
           Wm0_9,
           Wm1_0, Wm1_1, Wm1_2, Wm1_3, Wm1_4, Wm1_5, Wm1_6, Wm1_7, Wm1_8,
           Wm1_9,
           Wp_0, Wp_1, Wp_2, Wp_3, Wc, bc):
    xs = [x_domain0, x_domain1, x_domain2, x_domain3, x_domain4]
    eis = [ei_d0d0, ei_d1d1, ei_d2d2, ei_d3d3, ei_d1d0, ei_d0d1,
           ei_d1d2, ei_d2d1, ei_d1d3, ei_d3d1]
    wm0 = [Wm0_0, Wm0_1, Wm0_2, Wm0_3, Wm0_4, Wm0_5, Wm0_6, Wm0_7, Wm0_8,
           Wm0_9]
    wm1 = [Wm1_0, Wm1_1, Wm1_2, Wm1_3, Wm1_4, Wm1_5, Wm1_6, Wm1_7, Wm1_8,
           Wm1_9]
    wps = [Wp_0, Wp_1, Wp_2, Wp_3]

    adj_s = [jnp.zeros((_N, _N), jnp.float32).at[eis[i][0], eis[i][1]].add(1.0)
             for i in range(4)]
    adj = [adj_s[i].T for i in range(4)]
    adj += [jnp.zeros((_N, _N), jnp.float32).at[eis[i][1], eis[i][0]].add(1.0)
            for i in range(4, 10)]

    def embed_layer(h, Wself, Wms, adjs_by_type, types):
        self_p = [_pmm(h[k], Wself) for k in range(5)]
        ps = {i: _pmm(h[_EDGE[i][0]], Wms[i]) for i in types}
        out = []
        for k in range(5):
            inc = [i for i in types if _EDGE[i][1] == k]
            out.append(_fused(self_p[k],
                              [adjs_by_type[i] for i in inc],
                              [ps[i] for i in inc]))
        return out

    def gnn(types, attn_adj=None):
        h = xs
        for Wself, Wms in ((Wself0, wm0), (Wself1, wm1)):
            adjs = {i: (attn_adj[i] if attn_adj is not None and i in attn_adj
                        else adj[i]) for i in types}
            h = embed_layer(h, Wself, Wms, adjs, types)
        return h

    h1 = gnn([0, 1, 2, 3])

    gk = jax.random.key(42)
    attn_adj = {}
    for i in range(4):
        s, d = _EDGE[i]
        ei = eis[i]
        a = _pmm(h1[s], wps[i][:_H])[:, 0]
        b = _pmm(h1[d], wps[i][_H:])[:, 0]
        pred = jax.nn.sigmoid(a[ei[0]] + b[ei[1]])
        dense = jnp.zeros((_N, _N), jnp.float32).at[ei[0], ei[1]].add(pred)
        u = jax.random.uniform(jax.random.fold_in(gk, i), (_N, _N),
                               minval=1e-06, maxval=1.0 - 1e-06)
        g = -jnp.log(-jnp.log(u))
        attn_adj[i] = _topk_select(dense, g, adj_s[i]).T

    h2 = gnn(list(range(10)), attn_adj=attn_adj)

    pool = h2[1][index].reshape(1, -1)
    return jax.nn.softmax(pool @ Wc + bc, axis=1)
